# baseline (speedup 1.0000x reference)
import functools

import numpy as np
import jax
import jax.numpy as jnp
from jax import lax
from jax.experimental import pallas as pl
from jax.experimental.pallas import tpu as pltpu
from jax.experimental.pallas import tpu_sc as plsc

_L = 16
_F = 2
_T = 2 ** 19
_NMIN = 8
_NMAX = 2048
_BETA = float(np.exp(np.log(_NMAX / _NMIN) / (_L - 1)))
_RES = [int(np.floor(_NMIN * _BETA ** l)) for l in range(_L)]
_DENSE = [(r + 1) ** 4 <= _T for r in _RES]
_HK = [1,
       int(np.uint32(2654435761).astype(np.int32)),
       int(np.uint32(805459861).astype(np.int32)),
       int(np.uint32(3674653429).astype(np.int32))]

_NC = 2
_NS = 16
_NW = _NC * _NS

_C = 64
_G = _C // 16
_LH = _L // 2
_NIDXH = _LH * 16 * _C


def _enc_body(xyz_hbm, tab_hbm, enc_hbm, coords_v, idx_v, w_v, off_v, rows_v,
              enc_v, sem, *, n_pts):
    wid = lax.axis_index("s") * _NC + lax.axis_index("c")
    pts_per_w = n_pts // _NW
    nchunks = pts_per_w // _C
    iota = lax.iota(jnp.int32, 16)

    def chunk_body(k, carry):
        cbase = wid * pts_per_w + k * _C
        pltpu.sync_copy(xyz_hbm.at[pl.ds(cbase * 4, _C * 4)], coords_v)

        def grp_a(g, c2):
            p16 = g * 16
            cvec = []
            for d in range(4):
                v = plsc.load_gather(coords_v, [(p16 + iota) * 4 + d])
                if d < 3:
                    v = v * jnp.float32(1.0 / 3.0) + jnp.float32(0.5)
                cvec.append(v)
            for l in range(_L):
                res = _RES[l]
                dense = _DENSE[l]
                r1 = res + 1
                cont_lo, cont_hi, wlo, whi = [], [], [], []
                for d in range(4):
                    scaled = cvec[d] * jnp.float32(res)
                    pos = scaled.astype(jnp.int32)
                    frac = scaled - pos.astype(jnp.float32)
                    a = jnp.clip(pos, 0, res)
                    b = jnp.clip(pos + 1, 0, res)
                    kd = (r1 ** d) if dense else _HK[d]
                    if kd == 1:
                        cont_lo.append(a)
                        cont_hi.append(b)
                    else:
                        kd = jnp.int32(kd)
                        cont_lo.append(a * kd)
                        cont_hi.append(b * kd)
                    wlo.append(jnp.float32(1.0) - frac)
                    whi.append(frac)
                if dense:
                    comb = lambda u, v: u + v
                else:
                    comb = lambda u, v: u ^ v
                h01 = [comb(cont_lo[0], cont_lo[1]), comb(cont_hi[0], cont_lo[1]),
                       comb(cont_lo[0], cont_hi[1]), comb(cont_hi[0], cont_hi[1])]
                w01 = [wlo[0] * wlo[1], whi[0] * wlo[1],
                       wlo[0] * whi[1], whi[0] * whi[1]]
                h23 = [comb(cont_lo[2], cont_lo[3]), comb(cont_hi[2], cont_lo[3]),
                       comb(cont_lo[2], cont_hi[3]), comb(cont_hi[2], cont_hi[3])]
                w23 = [wlo[2] * wlo[3], whi[2] * wlo[3],
                       wlo[2] * whi[3], whi[2] * whi[3]]
                half = l // _LH
                lh = l % _LH
                for corner in range(16):
                    i01 = corner & 3
                    i23 = corner >> 2
                    idx = comb(h01[i01], h23[i23])
                    if not dense:
                        idx = idx & jnp.int32(_T - 1)
                    gidx = idx + jnp.int32(l * _T)
                    w = w01[i01] * w23[i23]
                    slot = (lh * 16 + corner) * _C + p16
                    idx_v[half, pl.ds(slot, 16)] = lax.shift_right_logical(gidx, 2)
                    off_v[half, pl.ds(slot, 16)] = lax.shift_left(gidx & jnp.int32(3), 1)
                    w_v[half, pl.ds(slot, 16)] = w
            return c2

        lax.fori_loop(0, _G, grp_a, 0)

        for half in range(2):
            pltpu.async_copy(tab_hbm.at[idx_v.at[half]], rows_v, sem).wait()

            def grp_c(g, c2):
                p16 = g * 16
                rows_pt = p16 + iota
                for lh in range(_LH):
                    l = half * _LH + lh
                    acc0 = jnp.zeros((16,), jnp.float32)
                    acc1 = jnp.zeros((16,), jnp.float32)
                    for corner in range(16):
                        slot = (lh * 16 + corner) * _C + p16
                        w = w_v[half, pl.ds(slot, 16)]
                        offv = off_v[half, pl.ds(slot, 16)]
                        rvec = slot + iota
                        f0 = plsc.load_gather(rows_v, [rvec, offv])
                        f1 = plsc.load_gather(rows_v, [rvec, offv + 1])
                        acc0 = acc0 + w * f0
                        acc1 = acc1 + w * f1
                    plsc.store_scatter(enc_v, [rows_pt, jnp.full((16,), 2 * l, jnp.int32)], acc0)
                    plsc.store_scatter(enc_v, [rows_pt, jnp.full((16,), 2 * l + 1, jnp.int32)], acc1)
                return c2

            lax.fori_loop(0, _G, grp_c, 0)

        pltpu.sync_copy(enc_v, enc_hbm.at[pl.ds(cbase, _C), :])
        return carry

    lax.fori_loop(0, nchunks, chunk_body, 0)


def _hash_encode_sc(xyz_flat, tab2d, n_pts):
    mesh = plsc.VectorSubcoreMesh(core_axis_name="c", subcore_axis_name="s",
                                  num_cores=_NC, num_subcores=_NS)
    return pl.kernel(
        functools.partial(_enc_body, n_pts=n_pts),
        out_type=jax.ShapeDtypeStruct((n_pts, 2 * _L), jnp.float32),
        mesh=mesh,
        compiler_params=pltpu.CompilerParams(needs_layout_passes=False,
                                             use_tc_tiling_on_sc=False),
        scratch_types=[
            pltpu.VMEM((4 * _C,), jnp.float32),
            pltpu.VMEM((2, _NIDXH), jnp.int32),
            pltpu.VMEM((2, _NIDXH), jnp.float32),
            pltpu.VMEM((2, _NIDXH), jnp.int32),
            pltpu.VMEM((_NIDXH, 8), jnp.float32),
            pltpu.VMEM((_C, 2 * _L), jnp.float32),
            pltpu.SemaphoreType.DMA,
        ],
    )(xyz_flat, tab2d)


def _mlp_body(enc_ref, dirs_ref, w0, w1, c0, c1, c2, out_ref):
    e = enc_ref[...]
    dn = (((1,), (0,)), ((), ()))
    h1 = jnp.maximum(
        lax.dot_general(e, w0[...], dn, preferred_element_type=jnp.float32),
        0.0)
    h = lax.dot_general(h1, w1[...], dn, preferred_element_type=jnp.float32)
    d = dirs_ref[...] * 2.0 - 1.0
    x = d[:, 0:1]
    y = d[:, 1:2]
    z = d[:, 2:3]
    x2, y2, z2 = x * x, y * y, z * z
    xy, yz, xz = x * y, y * z, x * z
    de = jnp.concatenate([
        0.28209479177387814 * jnp.ones_like(x),
        -0.48860251190291987 * y,
        0.48860251190291987 * z,
        -0.48860251190291987 * x,
        1.0925484305920792 * xy,
        -1.0925484305920792 * yz,
        0.94617469575755997 * z2 - 0.31539156525252005,
        -1.0925484305920792 * xz,
        0.54627421529603959 * (x2 - y2),
        0.59004358992664352 * y * (3.0 * x2 - y2),
        2.8906114426405538 * xy * z,
        0.45704579946446572 * y * (1.0 - 5.0 * z2),
        0.3731763325901154 * z * (5.0 * z2 - 3.0),
        0.45704579946446572 * x * (1.0 - 5.0 * z2),
        1.4453057213202769 * z * (x2 - y2),
        0.59004358992664352 * x * (x2 - 3.0 * y2),
    ], axis=1)
    cin = jnp.concatenate([de, h], axis=1)
    h2 = jnp.maximum(
        lax.dot_general(cin, c0[...], dn, preferred_element_type=jnp.float32),
        0.0)
    h3 = jnp.maximum(
        lax.dot_general(h2, c1[...], dn, preferred_element_type=jnp.float32),
        0.0)
    co = lax.dot_general(h3, c2[...], dn, preferred_element_type=jnp.float32)
    color = 1.0 / (1.0 + jnp.exp(-co))
    sigma = jnp.exp(h[:, 0:1])
    out_ref[...] = jnp.concatenate([color, sigma], axis=1)


def _mlp_tc(enc, dirs, W0, W1, C0, C1, C2, n_pts):
    bn = 2048
    grid = (n_pts // bn,)
    return pl.pallas_call(
        _mlp_body,
        grid=grid,
        in_specs=[
            pl.BlockSpec((bn, 2 * _L), lambda i: (i, 0)),
            pl.BlockSpec((bn, 3), lambda i: (i, 0)),
            pl.BlockSpec((32, 64), lambda i: (0, 0)),
            pl.BlockSpec((64, 16), lambda i: (0, 0)),
            pl.BlockSpec((32, 64), lambda i: (0, 0)),
            pl.BlockSpec((64, 64), lambda i: (0, 0)),
            pl.BlockSpec((64, 3), lambda i: (0, 0)),
        ],
        out_specs=pl.BlockSpec((bn, 4), lambda i: (i, 0)),
        out_shape=jax.ShapeDtypeStruct((n_pts, 4), jnp.float32),
    )(enc, dirs, W0, W1, C0, C1, C2)


def kernel(original_xyzs, dirs, tables, W0, W1, C0, C1, C2):
    n_pts = original_xyzs.shape[0]
    xyz_flat = original_xyzs.reshape(-1)
    tab8 = tables.reshape(_L * _T * _F // 8, 8)
    enc = _hash_encode_sc(xyz_flat, tab8, n_pts)
    out = _mlp_tc(enc, dirs, W0, W1, C0, C1, C2, n_pts)
    color = out[:, :3]
    sigma = out[:, 3]
    return (color, sigma)


# speedup vs baseline: 1.9323x; 1.9323x over previous
import functools

import numpy as np
import jax
import jax.numpy as jnp
from jax import lax
from jax.experimental import pallas as pl
from jax.experimental.pallas import tpu as pltpu
from jax.experimental.pallas import tpu_sc as plsc

_L = 16
_F = 2
_T = 2 ** 19
_NMIN = 8
_NMAX = 2048
_BETA = float(np.exp(np.log(_NMAX / _NMIN) / (_L - 1)))
_RES = [int(np.floor(_NMIN * _BETA ** l)) for l in range(_L)]
_DENSE = [(r + 1) ** 4 <= _T for r in _RES]
_HK = [1,
       int(np.uint32(2654435761).astype(np.int32)),
       int(np.uint32(805459861).astype(np.int32)),
       int(np.uint32(3674653429).astype(np.int32))]

_NC = 2
_NS = 16
_NW = _NC * _NS

_C = 32
_G = _C // 16
_LH = _L // 2
_NQ = _LH * 16 * _C
_NIDXH = 2 * _NQ


def _enc_body(xyz_hbm, tab_hbm, enc_hbm, coords_v, idx_v, w_v, off_v, rows_v,
              enc_v, sem, *, n_pts):
    wid = lax.axis_index("s") * _NC + lax.axis_index("c")
    pts_per_w = n_pts // _NW
    nchunks = pts_per_w // _C
    iota = lax.iota(jnp.int32, 16)

    def chunk_body(k, carry):
        cbase = wid * pts_per_w + k * _C
        pltpu.sync_copy(xyz_hbm.at[pl.ds(cbase * 4, _C * 4)], coords_v)

        def grp_a(g, c2):
            p16 = g * 16
            cvec = []
            for d in range(4):
                v = plsc.load_gather(coords_v, [(p16 + iota) * 4 + d])
                if d < 3:
                    v = v * jnp.float32(1.0 / 3.0) + jnp.float32(0.5)
                cvec.append(v)
            for l in range(_L):
                res = _RES[l]
                dense = _DENSE[l]
                r1 = res + 1
                cont_lo, cont_hi, wlo, whi = [], [], [], []
                for d in range(4):
                    scaled = cvec[d] * jnp.float32(res)
                    pos = scaled.astype(jnp.int32)
                    frac = scaled - pos.astype(jnp.float32)
                    a = jnp.clip(pos, 0, res)
                    b = jnp.clip(pos + 1, 0, res)
                    kd = (r1 ** d) if dense else _HK[d]
                    if kd == 1:
                        cont_lo.append(a)
                        cont_hi.append(b)
                    else:
                        kd = jnp.int32(kd)
                        cont_lo.append(a * kd)
                        cont_hi.append(b * kd)
                    wlo.append(jnp.float32(1.0) - frac)
                    whi.append(frac)
                if dense:
                    comb = lambda u, v: u + v
                else:
                    comb = lambda u, v: u ^ v
                h01 = [comb(cont_lo[0], cont_lo[1]), comb(cont_hi[0], cont_lo[1]),
                       comb(cont_lo[0], cont_hi[1]), comb(cont_hi[0], cont_hi[1])]
                w01 = [wlo[0] * wlo[1], whi[0] * wlo[1],
                       wlo[0] * whi[1], whi[0] * whi[1]]
                h23 = [comb(cont_lo[2], cont_lo[3]), comb(cont_hi[2], cont_lo[3]),
                       comb(cont_lo[2], cont_hi[3]), comb(cont_hi[2], cont_hi[3])]
                w23 = [wlo[2] * wlo[3], whi[2] * wlo[3],
                       wlo[2] * whi[3], whi[2] * whi[3]]
                half = l // _LH
                lh = l % _LH
                for corner in range(16):
                    i01 = corner & 3
                    i23 = corner >> 2
                    idx = comb(h01[i01], h23[i23])
                    if not dense:
                        idx = idx & jnp.int32(_T - 1)
                    w = w01[i01] * w23[i23]
                    slot = (lh * 16 + corner) * _C + p16
                    w0 = (jnp.int32(l * 2 * _T)
                          + lax.shift_left(lax.shift_right_logical(idx, 7), 8)
                          + (idx & jnp.int32(127)))
                    q0 = lax.shift_right_logical(w0, 3)
                    idx_v[half, pl.ds(slot, 16)] = q0
                    idx_v[half, pl.ds(_NQ + slot, 16)] = q0 + jnp.int32(16)
                    off_v[half, pl.ds(slot, 16)] = idx & jnp.int32(7)
                    w_v[half, pl.ds(slot, 16)] = w
            return c2

        lax.fori_loop(0, _G, grp_a, 0)

        for half in range(2):
            pltpu.async_copy(tab_hbm.at[idx_v.at[half]], rows_v, sem).wait()

            def grp_c(g, c2):
                p16 = g * 16
                rows_pt = p16 + iota
                for lh in range(_LH):
                    l = half * _LH + lh
                    acc0 = jnp.zeros((16,), jnp.float32)
                    acc1 = jnp.zeros((16,), jnp.float32)
                    for corner in range(16):
                        slot = (lh * 16 + corner) * _C + p16
                        w = w_v[half, pl.ds(slot, 16)]
                        offv = off_v[half, pl.ds(slot, 16)]
                        rvec = slot + iota
                        f0 = plsc.load_gather(rows_v, [rvec, offv])
                        f1 = plsc.load_gather(rows_v, [rvec + jnp.int32(_NQ), offv])
                        acc0 = acc0 + w * f0
                        acc1 = acc1 + w * f1
                    plsc.store_scatter(enc_v, [rows_pt, jnp.full((16,), 2 * l, jnp.int32)], acc0)
                    plsc.store_scatter(enc_v, [rows_pt, jnp.full((16,), 2 * l + 1, jnp.int32)], acc1)
                return c2

            lax.fori_loop(0, _G, grp_c, 0)

        pltpu.sync_copy(enc_v, enc_hbm.at[pl.ds(cbase, _C), :])
        return carry

    lax.fori_loop(0, nchunks, chunk_body, 0)


def _hash_encode_sc(xyz_flat, tab2d, n_pts):
    mesh = plsc.VectorSubcoreMesh(core_axis_name="c", subcore_axis_name="s",
                                  num_cores=_NC, num_subcores=_NS)
    return pl.kernel(
        functools.partial(_enc_body, n_pts=n_pts),
        out_type=jax.ShapeDtypeStruct((n_pts, 2 * _L), jnp.float32),
        mesh=mesh,
        compiler_params=pltpu.CompilerParams(needs_layout_passes=False,
                                             use_tc_tiling_on_sc=False),
        scratch_types=[
            pltpu.VMEM((4 * _C,), jnp.float32),
            pltpu.VMEM((2, _NIDXH), jnp.int32),
            pltpu.VMEM((2, _NQ), jnp.float32),
            pltpu.VMEM((2, _NQ), jnp.int32),
            pltpu.VMEM((_NIDXH, 8), jnp.float32),
            pltpu.VMEM((_C, 2 * _L), jnp.float32),
            pltpu.SemaphoreType.DMA,
        ],
    )(xyz_flat, tab2d)


def _mlp_body(enc_ref, dirs_ref, w0, w1, c0, c1, c2, out_ref):
    e = enc_ref[...]
    dn = (((1,), (0,)), ((), ()))
    h1 = jnp.maximum(
        lax.dot_general(e, w0[...], dn, preferred_element_type=jnp.float32),
        0.0)
    h = lax.dot_general(h1, w1[...], dn, preferred_element_type=jnp.float32)
    d = dirs_ref[...] * 2.0 - 1.0
    x = d[:, 0:1]
    y = d[:, 1:2]
    z = d[:, 2:3]
    x2, y2, z2 = x * x, y * y, z * z
    xy, yz, xz = x * y, y * z, x * z
    de = jnp.concatenate([
        0.28209479177387814 * jnp.ones_like(x),
        -0.48860251190291987 * y,
        0.48860251190291987 * z,
        -0.48860251190291987 * x,
        1.0925484305920792 * xy,
        -1.0925484305920792 * yz,
        0.94617469575755997 * z2 - 0.31539156525252005,
        -1.0925484305920792 * xz,
        0.54627421529603959 * (x2 - y2),
        0.59004358992664352 * y * (3.0 * x2 - y2),
        2.8906114426405538 * xy * z,
        0.45704579946446572 * y * (1.0 - 5.0 * z2),
        0.3731763325901154 * z * (5.0 * z2 - 3.0),
        0.45704579946446572 * x * (1.0 - 5.0 * z2),
        1.4453057213202769 * z * (x2 - y2),
        0.59004358992664352 * x * (x2 - 3.0 * y2),
    ], axis=1)
    cin = jnp.concatenate([de, h], axis=1)
    h2 = jnp.maximum(
        lax.dot_general(cin, c0[...], dn, preferred_element_type=jnp.float32),
        0.0)
    h3 = jnp.maximum(
        lax.dot_general(h2, c1[...], dn, preferred_element_type=jnp.float32),
        0.0)
    co = lax.dot_general(h3, c2[...], dn, preferred_element_type=jnp.float32)
    color = 1.0 / (1.0 + jnp.exp(-co))
    sigma = jnp.exp(h[:, 0:1])
    out_ref[...] = jnp.concatenate([color, sigma], axis=1)


def _mlp_tc(enc, dirs, W0, W1, C0, C1, C2, n_pts):
    bn = 2048
    grid = (n_pts // bn,)
    return pl.pallas_call(
        _mlp_body,
        grid=grid,
        in_specs=[
            pl.BlockSpec((bn, 2 * _L), lambda i: (i, 0)),
            pl.BlockSpec((bn, 3), lambda i: (i, 0)),
            pl.BlockSpec((32, 64), lambda i: (0, 0)),
            pl.BlockSpec((64, 16), lambda i: (0, 0)),
            pl.BlockSpec((32, 64), lambda i: (0, 0)),
            pl.BlockSpec((64, 64), lambda i: (0, 0)),
            pl.BlockSpec((64, 3), lambda i: (0, 0)),
        ],
        out_specs=pl.BlockSpec((bn, 4), lambda i: (i, 0)),
        out_shape=jax.ShapeDtypeStruct((n_pts, 4), jnp.float32),
    )(enc, dirs, W0, W1, C0, C1, C2)


def kernel(original_xyzs, dirs, tables, W0, W1, C0, C1, C2):
    n_pts = original_xyzs.shape[0]
    xyz_flat = original_xyzs.reshape(-1)
    tab8 = (tables.reshape(_L, _T // 128, 128, _F)
            .swapaxes(2, 3)
            .reshape(_L * _T * _F // 8, 8))
    enc = _hash_encode_sc(xyz_flat, tab8, n_pts)
    out = _mlp_tc(enc, dirs, W0, W1, C0, C1, C2, n_pts)
    color = out[:, :3]
    sigma = out[:, 3]
    return (color, sigma)


# speedup vs baseline: 2.2991x; 1.1898x over previous
import functools

import numpy as np
import jax
import jax.numpy as jnp
from jax import lax
from jax.experimental import pallas as pl
from jax.experimental.pallas import tpu as pltpu
from jax.experimental.pallas import tpu_sc as plsc

_L = 16
_F = 2
_T = 2 ** 19
_NMIN = 8
_NMAX = 2048
_BETA = float(np.exp(np.log(_NMAX / _NMIN) / (_L - 1)))
_RES = [int(np.floor(_NMIN * _BETA ** l)) for l in range(_L)]
_DENSE = [(r + 1) ** 4 <= _T for r in _RES]
_HK = [1,
       int(np.uint32(2654435761).astype(np.int32)),
       int(np.uint32(805459861).astype(np.int32)),
       int(np.uint32(3674653429).astype(np.int32))]

_NC = 2
_NS = 16
_NW = _NC * _NS

_C = 32
_G = _C // 16
_NSEG = 4
_LQ = _L // _NSEG
_NQ = _LQ * 16 * _C
_NIDXH = 2 * _NQ


def _enc_body(xyz_hbm, tab_hbm, enc_hbm, coords_v, idx_v, w_v, off_v, rows_v,
              enc_v, sem0, sem1, *, n_pts):
    wid = lax.axis_index("s") * _NC + lax.axis_index("c")
    pts_per_w = n_pts // _NW
    nchunks = pts_per_w // _C
    iota = lax.iota(jnp.int32, 16)
    sems = (sem0, sem1)

    def phase_a(seg):
        def grp_a(g, c2):
            p16 = g * 16
            cvec = []
            for d in range(4):
                v = plsc.load_gather(coords_v, [(p16 + iota) * 4 + d])
                if d < 3:
                    v = v * jnp.float32(1.0 / 3.0) + jnp.float32(0.5)
                cvec.append(v)
            for lq in range(_LQ):
                l = seg * _LQ + lq
                res = _RES[l]
                dense = _DENSE[l]
                r1 = res + 1
                cont_lo, cont_hi, wlo, whi = [], [], [], []
                for d in range(4):
                    scaled = cvec[d] * jnp.float32(res)
                    pos = scaled.astype(jnp.int32)
                    frac = scaled - pos.astype(jnp.float32)
                    a = jnp.clip(pos, 0, res)
                    b = jnp.clip(pos + 1, 0, res)
                    kd = (r1 ** d) if dense else _HK[d]
                    if kd == 1:
                        cont_lo.append(a)
                        cont_hi.append(b)
                    else:
                        kd = jnp.int32(kd)
                        cont_lo.append(a * kd)
                        cont_hi.append(b * kd)
                    wlo.append(jnp.float32(1.0) - frac)
                    whi.append(frac)
                if dense:
                    comb = lambda u, v: u + v
                else:
                    comb = lambda u, v: u ^ v
                h01 = [comb(cont_lo[0], cont_lo[1]), comb(cont_hi[0], cont_lo[1]),
                       comb(cont_lo[0], cont_hi[1]), comb(cont_hi[0], cont_hi[1])]
                w01 = [wlo[0] * wlo[1], whi[0] * wlo[1],
                       wlo[0] * whi[1], whi[0] * whi[1]]
                h23 = [comb(cont_lo[2], cont_lo[3]), comb(cont_hi[2], cont_lo[3]),
                       comb(cont_lo[2], cont_hi[3]), comb(cont_hi[2], cont_hi[3])]
                w23 = [wlo[2] * wlo[3], whi[2] * wlo[3],
                       wlo[2] * whi[3], whi[2] * whi[3]]
                for corner in range(16):
                    i01 = corner & 3
                    i23 = corner >> 2
                    idx = comb(h01[i01], h23[i23])
                    if not dense:
                        idx = idx & jnp.int32(_T - 1)
                    w = w01[i01] * w23[i23]
                    slot = (lq * 16 + corner) * _C + p16
                    w0 = (jnp.int32(l * 2 * _T)
                          + lax.shift_left(lax.shift_right_logical(idx, 7), 8)
                          + (idx & jnp.int32(127)))
                    q0 = lax.shift_right_logical(w0, 3)
                    idx_v[seg, pl.ds(slot, 16)] = q0
                    idx_v[seg, pl.ds(_NQ + slot, 16)] = q0 + jnp.int32(16)
                    off_v[seg, pl.ds(slot, 16)] = idx & jnp.int32(7)
                    w_v[seg, pl.ds(slot, 16)] = w
            return c2

        lax.fori_loop(0, _G, grp_a, 0)

    def fire(seg):
        return pltpu.async_copy(tab_hbm.at[idx_v.at[seg]], rows_v.at[seg % 2],
                                sems[seg % 2])

    def phase_c(seg):
        rows = rows_v.at[seg % 2]

        def grp_c(g, c2):
            p16 = g * 16
            rows_pt = p16 + iota
            for lq in range(_LQ):
                l = seg * _LQ + lq
                acc0 = jnp.zeros((16,), jnp.float32)
                acc1 = jnp.zeros((16,), jnp.float32)
                for corner in range(16):
                    slot = (lq * 16 + corner) * _C + p16
                    w = w_v[seg, pl.ds(slot, 16)]
                    offv = off_v[seg, pl.ds(slot, 16)]
                    rvec = slot + iota
                    f0 = plsc.load_gather(rows, [rvec, offv])
                    f1 = plsc.load_gather(rows, [rvec + jnp.int32(_NQ), offv])
                    acc0 = acc0 + w * f0
                    acc1 = acc1 + w * f1
                plsc.store_scatter(enc_v, [rows_pt, jnp.full((16,), 2 * l, jnp.int32)], acc0)
                plsc.store_scatter(enc_v, [rows_pt, jnp.full((16,), 2 * l + 1, jnp.int32)], acc1)
            return c2

        lax.fori_loop(0, _G, grp_c, 0)

    def chunk_body(k, carry):
        cbase = wid * pts_per_w + k * _C
        pltpu.sync_copy(xyz_hbm.at[pl.ds(cbase * 4, _C * 4)], coords_v)
        phase_a(0)
        d0 = fire(0)
        phase_a(1)
        d1 = fire(1)
        d0.wait()
        phase_c(0)
        phase_a(2)
        d2 = fire(2)
        d1.wait()
        phase_c(1)
        phase_a(3)
        d3 = fire(3)
        d2.wait()
        phase_c(2)
        d3.wait()
        phase_c(3)
        pltpu.sync_copy(enc_v, enc_hbm.at[pl.ds(cbase, _C), :])
        return carry

    lax.fori_loop(0, nchunks, chunk_body, 0)


def _hash_encode_sc(xyz_flat, tab2d, n_pts):
    mesh = plsc.VectorSubcoreMesh(core_axis_name="c", subcore_axis_name="s",
                                  num_cores=_NC, num_subcores=_NS)
    return pl.kernel(
        functools.partial(_enc_body, n_pts=n_pts),
        out_type=jax.ShapeDtypeStruct((n_pts, 2 * _L), jnp.float32),
        mesh=mesh,
        compiler_params=pltpu.CompilerParams(needs_layout_passes=False,
                                             use_tc_tiling_on_sc=False),
        scratch_types=[
            pltpu.VMEM((4 * _C,), jnp.float32),
            pltpu.VMEM((_NSEG, _NIDXH), jnp.int32),
            pltpu.VMEM((_NSEG, _NQ), jnp.float32),
            pltpu.VMEM((_NSEG, _NQ), jnp.int32),
            pltpu.VMEM((2, _NIDXH, 8), jnp.float32),
            pltpu.VMEM((_C, 2 * _L), jnp.float32),
            pltpu.SemaphoreType.DMA,
            pltpu.SemaphoreType.DMA,
        ],
    )(xyz_flat, tab2d)


def _mlp_body(enc_ref, dirs_ref, w0, w1, c0, c1, c2, out_ref):
    e = enc_ref[...]
    dn = (((1,), (0,)), ((), ()))
    h1 = jnp.maximum(
        lax.dot_general(e, w0[...], dn, preferred_element_type=jnp.float32),
        0.0)
    h = lax.dot_general(h1, w1[...], dn, preferred_element_type=jnp.float32)
    d = dirs_ref[...] * 2.0 - 1.0
    x = d[:, 0:1]
    y = d[:, 1:2]
    z = d[:, 2:3]
    x2, y2, z2 = x * x, y * y, z * z
    xy, yz, xz = x * y, y * z, x * z
    de = jnp.concatenate([
        0.28209479177387814 * jnp.ones_like(x),
        -0.48860251190291987 * y,
        0.48860251190291987 * z,
        -0.48860251190291987 * x,
        1.0925484305920792 * xy,
        -1.0925484305920792 * yz,
        0.94617469575755997 * z2 - 0.31539156525252005,
        -1.0925484305920792 * xz,
        0.54627421529603959 * (x2 - y2),
        0.59004358992664352 * y * (3.0 * x2 - y2),
        2.8906114426405538 * xy * z,
        0.45704579946446572 * y * (1.0 - 5.0 * z2),
        0.3731763325901154 * z * (5.0 * z2 - 3.0),
        0.45704579946446572 * x * (1.0 - 5.0 * z2),
        1.4453057213202769 * z * (x2 - y2),
        0.59004358992664352 * x * (x2 - 3.0 * y2),
    ], axis=1)
    cin = jnp.concatenate([de, h], axis=1)
    h2 = jnp.maximum(
        lax.dot_general(cin, c0[...], dn, preferred_element_type=jnp.float32),
        0.0)
    h3 = jnp.maximum(
        lax.dot_general(h2, c1[...], dn, preferred_element_type=jnp.float32),
        0.0)
    co = lax.dot_general(h3, c2[...], dn, preferred_element_type=jnp.float32)
    color = 1.0 / (1.0 + jnp.exp(-co))
    sigma = jnp.exp(h[:, 0:1])
    out_ref[...] = jnp.concatenate([color, sigma], axis=1)


def _mlp_tc(enc, dirs, W0, W1, C0, C1, C2, n_pts):
    bn = 2048
    grid = (n_pts // bn,)
    return pl.pallas_call(
        _mlp_body,
        grid=grid,
        in_specs=[
            pl.BlockSpec((bn, 2 * _L), lambda i: (i, 0)),
            pl.BlockSpec((bn, 3), lambda i: (i, 0)),
            pl.BlockSpec((32, 64), lambda i: (0, 0)),
            pl.BlockSpec((64, 16), lambda i: (0, 0)),
            pl.BlockSpec((32, 64), lambda i: (0, 0)),
            pl.BlockSpec((64, 64), lambda i: (0, 0)),
            pl.BlockSpec((64, 3), lambda i: (0, 0)),
        ],
        out_specs=pl.BlockSpec((bn, 4), lambda i: (i, 0)),
        out_shape=jax.ShapeDtypeStruct((n_pts, 4), jnp.float32),
    )(enc, dirs, W0, W1, C0, C1, C2)


def kernel(original_xyzs, dirs, tables, W0, W1, C0, C1, C2):
    n_pts = original_xyzs.shape[0]
    xyz_flat = original_xyzs.reshape(-1)
    tab8 = (tables.reshape(_L, _T // 128, 128, _F)
            .swapaxes(2, 3)
            .reshape(_L * _T * _F // 8, 8))
    enc = _hash_encode_sc(xyz_flat, tab8, n_pts)
    out = _mlp_tc(enc, dirs, W0, W1, C0, C1, C2, n_pts)
    color = out[:, :3]
    sigma = out[:, 3]
    return (color, sigma)


# speedup vs baseline: 2.4887x; 1.0825x over previous
import functools

import numpy as np
import jax
import jax.numpy as jnp
from jax import lax
from jax.experimental import pallas as pl
from jax.experimental.pallas import tpu as pltpu
from jax.experimental.pallas import tpu_sc as plsc

_L = 16
_F = 2
_T = 2 ** 19
_NMIN = 8
_NMAX = 2048
_BETA = float(np.exp(np.log(_NMAX / _NMIN) / (_L - 1)))
_RES = [int(np.floor(_NMIN * _BETA ** l)) for l in range(_L)]
_DENSE = [(r + 1) ** 4 <= _T for r in _RES]
_HK = [1,
       int(np.uint32(2654435761).astype(np.int32)),
       int(np.uint32(805459861).astype(np.int32)),
       int(np.uint32(3674653429).astype(np.int32))]

_NC = 2
_NS = 16
_NW = _NC * _NS

_C = 32
_G = _C // 16
_NSEG = 4
_LQ = _L // _NSEG
_NQ = _LQ * 16 * _C
_NIDXH = 2 * _NQ


def _enc_body(xyz_hbm, tab_hbm, enc_hbm, coords_v, idx_v, w_v, off_v, rows_v,
              enc_v, sem0, sem1, *, n_pts):
    wid = lax.axis_index("s") * _NC + lax.axis_index("c")
    pts_per_w = n_pts // _NW
    nchunks = pts_per_w // _C
    iota = lax.iota(jnp.int32, 16)
    sems = (sem0, sem1)

    def phase_a(seg):
        def grp_a(g, c2):
            p16 = g * 16
            cvec = []
            for d in range(4):
                v = plsc.load_gather(coords_v, [(p16 + iota) * 4 + d])
                if d < 3:
                    v = v * jnp.float32(1.0 / 3.0) + jnp.float32(0.5)
                cvec.append(v)
            for lq in range(_LQ):
                l = seg * _LQ + lq
                res = _RES[l]
                dense = _DENSE[l]
                r1 = res + 1
                cont_lo, cont_hi, wlo, whi = [], [], [], []
                for d in range(4):
                    scaled = cvec[d] * jnp.float32(res)
                    pos = scaled.astype(jnp.int32)
                    frac = scaled - pos.astype(jnp.float32)
                    a = jnp.clip(pos, 0, res)
                    b = jnp.clip(pos + 1, 0, res)
                    kd = (r1 ** d) if dense else _HK[d]
                    if kd == 1:
                        cont_lo.append(a)
                        cont_hi.append(b)
                    else:
                        kd = jnp.int32(kd)
                        cont_lo.append(a * kd)
                        cont_hi.append(b * kd)
                    wlo.append(jnp.float32(1.0) - frac)
                    whi.append(frac)
                if dense:
                    comb = lambda u, v: u + v
                else:
                    comb = lambda u, v: u ^ v
                h01 = [comb(cont_lo[0], cont_lo[1]), comb(cont_hi[0], cont_lo[1]),
                       comb(cont_lo[0], cont_hi[1]), comb(cont_hi[0], cont_hi[1])]
                w01 = [wlo[0] * wlo[1], whi[0] * wlo[1],
                       wlo[0] * whi[1], whi[0] * whi[1]]
                h23 = [comb(cont_lo[2], cont_lo[3]), comb(cont_hi[2], cont_lo[3]),
                       comb(cont_lo[2], cont_hi[3]), comb(cont_hi[2], cont_hi[3])]
                w23 = [wlo[2] * wlo[3], whi[2] * wlo[3],
                       wlo[2] * whi[3], whi[2] * whi[3]]
                for corner in range(16):
                    i01 = corner & 3
                    i23 = corner >> 2
                    idx = comb(h01[i01], h23[i23])
                    if not dense:
                        idx = idx & jnp.int32(_T - 1)
                    w = w01[i01] * w23[i23]
                    slot = (lq * 16 + corner) * _C + p16
                    w0 = (jnp.int32(l * 2 * _T)
                          + lax.shift_left(lax.shift_right_logical(idx, 7), 8)
                          + (idx & jnp.int32(127)))
                    q0 = lax.shift_right_logical(w0, 3)
                    idx_v[seg, pl.ds(slot, 16)] = q0
                    idx_v[seg, pl.ds(_NQ + slot, 16)] = q0 + jnp.int32(16)
                    off_v[seg, pl.ds(slot, 16)] = idx & jnp.int32(7)
                    w_v[seg, pl.ds(slot, 16)] = w
            return c2

        lax.fori_loop(0, _G, grp_a, 0)

    def fire(seg):
        return pltpu.async_copy(tab_hbm.at[idx_v.at[seg]], rows_v.at[seg % 2],
                                sems[seg % 2])

    def phase_c(seg):
        rows = rows_v.at[seg % 2]

        def grp_c(g, c2):
            p16 = g * 16
            rows_pt = p16 + iota
            for lq in range(_LQ):
                l = seg * _LQ + lq
                acc0 = jnp.zeros((16,), jnp.float32)
                acc1 = jnp.zeros((16,), jnp.float32)
                for corner in range(16):
                    slot = (lq * 16 + corner) * _C + p16
                    w = w_v[seg, pl.ds(slot, 16)]
                    offv = off_v[seg, pl.ds(slot, 16)]
                    rvec = slot + iota
                    f0 = plsc.load_gather(rows, [rvec, offv])
                    f1 = plsc.load_gather(rows, [rvec + jnp.int32(_NQ), offv])
                    acc0 = acc0 + w * f0
                    acc1 = acc1 + w * f1
                plsc.store_scatter(enc_v, [rows_pt, jnp.full((16,), 2 * l, jnp.int32)], acc0)
                plsc.store_scatter(enc_v, [rows_pt, jnp.full((16,), 2 * l + 1, jnp.int32)], acc1)
            return c2

        lax.fori_loop(0, _G, grp_c, 0)

    def chunk_body(k, carry):
        cbase = wid * pts_per_w + k * _C
        pltpu.sync_copy(xyz_hbm.at[pl.ds(cbase * 4, _C * 4)], coords_v)
        phase_a(0)
        d0 = fire(0)
        phase_a(1)
        d1 = fire(1)
        d0.wait()
        phase_c(0)
        phase_a(2)
        d2 = fire(2)
        d1.wait()
        phase_c(1)
        phase_a(3)
        d3 = fire(3)
        d2.wait()
        phase_c(2)
        d3.wait()
        phase_c(3)
        pltpu.sync_copy(enc_v, enc_hbm.at[pl.ds(cbase, _C), :])
        return carry

    lax.fori_loop(0, nchunks, chunk_body, 0)


def _hash_encode_sc(xyz_flat, tab2d, n_pts):
    mesh = plsc.VectorSubcoreMesh(core_axis_name="c", subcore_axis_name="s",
                                  num_cores=_NC, num_subcores=_NS)
    return pl.kernel(
        functools.partial(_enc_body, n_pts=n_pts),
        out_type=jax.ShapeDtypeStruct((n_pts, 2 * _L), jnp.float32),
        mesh=mesh,
        compiler_params=pltpu.CompilerParams(needs_layout_passes=False,
                                             use_tc_tiling_on_sc=False),
        scratch_types=[
            pltpu.VMEM((4 * _C,), jnp.float32),
            pltpu.VMEM((_NSEG, _NIDXH), jnp.int32),
            pltpu.VMEM((_NSEG, _NQ), jnp.float32),
            pltpu.VMEM((_NSEG, _NQ), jnp.int32),
            pltpu.VMEM((2, _NIDXH, 8), jnp.float32),
            pltpu.VMEM((_C, 2 * _L), jnp.float32),
            pltpu.SemaphoreType.DMA,
            pltpu.SemaphoreType.DMA,
        ],
    )(xyz_flat, tab2d)


def _mlp_body(enc_ref, dirs_ref, w0, w1, c0, c1, c2, out_ref):
    e = enc_ref[...]
    dn = (((1,), (0,)), ((), ()))
    h1 = jnp.maximum(
        lax.dot_general(e, w0[...], dn, preferred_element_type=jnp.float32),
        0.0)
    h = lax.dot_general(h1, w1[...], dn, preferred_element_type=jnp.float32)
    d = dirs_ref[...] * 2.0 - 1.0
    x = d[:, 0:1]
    y = d[:, 1:2]
    z = d[:, 2:3]
    x2, y2, z2 = x * x, y * y, z * z
    xy, yz, xz = x * y, y * z, x * z
    de = jnp.concatenate([
        0.28209479177387814 * jnp.ones_like(x),
        -0.48860251190291987 * y,
        0.48860251190291987 * z,
        -0.48860251190291987 * x,
        1.0925484305920792 * xy,
        -1.0925484305920792 * yz,
        0.94617469575755997 * z2 - 0.31539156525252005,
        -1.0925484305920792 * xz,
        0.54627421529603959 * (x2 - y2),
        0.59004358992664352 * y * (3.0 * x2 - y2),
        2.8906114426405538 * xy * z,
        0.45704579946446572 * y * (1.0 - 5.0 * z2),
        0.3731763325901154 * z * (5.0 * z2 - 3.0),
        0.45704579946446572 * x * (1.0 - 5.0 * z2),
        1.4453057213202769 * z * (x2 - y2),
        0.59004358992664352 * x * (x2 - 3.0 * y2),
    ], axis=1)
    cin = jnp.concatenate([de, h], axis=1)
    h2 = jnp.maximum(
        lax.dot_general(cin, c0[...], dn, preferred_element_type=jnp.float32),
        0.0)
    h3 = jnp.maximum(
        lax.dot_general(h2, c1[...], dn, preferred_element_type=jnp.float32),
        0.0)
    co = lax.dot_general(h3, c2[...], dn, preferred_element_type=jnp.float32)
    color = 1.0 / (1.0 + jnp.exp(-co))
    sigma = jnp.exp(h[:, 0:1])
    out_ref[...] = jnp.concatenate([color, sigma], axis=1)


def _mlp_tc(enc, dirs, W0, W1, C0, C1, C2, n_pts):
    bn = 2048
    grid = (n_pts // bn,)
    return pl.pallas_call(
        _mlp_body,
        grid=grid,
        in_specs=[
            pl.BlockSpec((bn, 2 * _L), lambda i: (i, 0)),
            pl.BlockSpec((bn, 3), lambda i: (i, 0)),
            pl.BlockSpec((32, 64), lambda i: (0, 0)),
            pl.BlockSpec((64, 16), lambda i: (0, 0)),
            pl.BlockSpec((32, 64), lambda i: (0, 0)),
            pl.BlockSpec((64, 64), lambda i: (0, 0)),
            pl.BlockSpec((64, 3), lambda i: (0, 0)),
        ],
        out_specs=pl.BlockSpec((bn, 4), lambda i: (i, 0)),
        out_shape=jax.ShapeDtypeStruct((n_pts, 4), jnp.float32),
    )(enc, dirs, W0, W1, C0, C1, C2)


def kernel(original_xyzs, dirs, tables, W0, W1, C0, C1, C2):
    n_pts = original_xyzs.shape[0]
    tab8 = (tables.reshape(_L, _T // 128, 128, _F)
            .swapaxes(2, 3)
            .reshape(_L * _T * _F // 8, 8))
    nh = n_pts // 2
    outs = []
    for h in range(2):
        xyz_h = original_xyzs[h * nh:(h + 1) * nh].reshape(-1)
        enc_h = _hash_encode_sc(xyz_h, tab8, nh)
        outs.append(_mlp_tc(enc_h, dirs[h * nh:(h + 1) * nh],
                            W0, W1, C0, C1, C2, nh))
    out = jnp.concatenate(outs, axis=0)
    color = out[:, :3]
    sigma = out[:, 3]
    return (color, sigma)


# speedup vs baseline: 2.5348x; 1.0185x over previous
import functools

import numpy as np
import jax
import jax.numpy as jnp
from jax import lax
from jax.experimental import pallas as pl
from jax.experimental.pallas import tpu as pltpu
from jax.experimental.pallas import tpu_sc as plsc

_L = 16
_F = 2
_T = 2 ** 19
_NMIN = 8
_NMAX = 2048
_BETA = float(np.exp(np.log(_NMAX / _NMIN) / (_L - 1)))
_RES = [int(np.floor(_NMIN * _BETA ** l)) for l in range(_L)]
_DENSE = [(r + 1) ** 4 <= _T for r in _RES]
_HK = [1,
       int(np.uint32(2654435761).astype(np.int32)),
       int(np.uint32(805459861).astype(np.int32)),
       int(np.uint32(3674653429).astype(np.int32))]

_NC = 2
_NS = 16
_NW = _NC * _NS

_C = 32
_G = _C // 16
_NSEG = 4
_LQ = _L // _NSEG
_NQ = _LQ * 16 * _C
_NIDXH = 2 * _NQ


def _enc_body(xyz_hbm, tab_hbm, enc_hbm, coords_v, idx_v, w_v, off_v, rows_v,
              enc_v, sem0, sem1, *, n_pts):
    wid = lax.axis_index("s") * _NC + lax.axis_index("c")
    pts_per_w = n_pts // _NW
    nchunks = pts_per_w // _C
    iota = lax.iota(jnp.int32, 16)
    sems = (sem0, sem1)

    def phase_a(seg):
        def grp_a(g, c2):
            p16 = g * 16
            cvec = []
            for d in range(4):
                v = plsc.load_gather(coords_v, [(p16 + iota) * 4 + d])
                if d < 3:
                    v = v * jnp.float32(1.0 / 3.0) + jnp.float32(0.5)
                cvec.append(v)
            for lq in range(_LQ):
                l = seg * _LQ + lq
                res = _RES[l]
                dense = _DENSE[l]
                r1 = res + 1
                cont_lo, cont_hi, wlo, whi = [], [], [], []
                for d in range(4):
                    scaled = cvec[d] * jnp.float32(res)
                    pos = scaled.astype(jnp.int32)
                    frac = scaled - pos.astype(jnp.float32)
                    a = jnp.clip(pos, 0, res)
                    b = jnp.clip(pos + 1, 0, res)
                    kd = (r1 ** d) if dense else _HK[d]
                    if kd == 1:
                        cont_lo.append(a)
                        cont_hi.append(b)
                    else:
                        kd = jnp.int32(kd)
                        cont_lo.append(a * kd)
                        cont_hi.append(b * kd)
                    wlo.append(jnp.float32(1.0) - frac)
                    whi.append(frac)
                if dense:
                    comb = lambda u, v: u + v
                else:
                    comb = lambda u, v: u ^ v
                h01 = [comb(cont_lo[0], cont_lo[1]), comb(cont_hi[0], cont_lo[1]),
                       comb(cont_lo[0], cont_hi[1]), comb(cont_hi[0], cont_hi[1])]
                w01 = [wlo[0] * wlo[1], whi[0] * wlo[1],
                       wlo[0] * whi[1], whi[0] * whi[1]]
                h23 = [comb(cont_lo[2], cont_lo[3]), comb(cont_hi[2], cont_lo[3]),
                       comb(cont_lo[2], cont_hi[3]), comb(cont_hi[2], cont_hi[3])]
                w23 = [wlo[2] * wlo[3], whi[2] * wlo[3],
                       wlo[2] * whi[3], whi[2] * whi[3]]
                for corner in range(16):
                    i01 = corner & 3
                    i23 = corner >> 2
                    idx = comb(h01[i01], h23[i23])
                    if not dense:
                        idx = idx & jnp.int32(_T - 1)
                    w = w01[i01] * w23[i23]
                    slot = (lq * 16 + corner) * _C + p16
                    w0 = (jnp.int32(l * 2 * _T)
                          + lax.shift_left(lax.shift_right_logical(idx, 7), 8)
                          + (idx & jnp.int32(127)))
                    q0 = lax.shift_right_logical(w0, 3)
                    idx_v[seg, pl.ds(slot, 16)] = q0
                    idx_v[seg, pl.ds(_NQ + slot, 16)] = q0 + jnp.int32(16)
                    off_v[seg, pl.ds(slot, 16)] = idx & jnp.int32(7)
                    w_v[seg, pl.ds(slot, 16)] = w
            return c2

        lax.fori_loop(0, _G, grp_a, 0)

    def fire(seg):
        return pltpu.async_copy(tab_hbm.at[idx_v.at[seg]], rows_v.at[seg % 2],
                                sems[seg % 2])

    def phase_c(seg):
        rows = rows_v.at[seg % 2]

        def grp_c(g, c2):
            p16 = g * 16
            rows_pt = p16 + iota
            for lq in range(_LQ):
                l = seg * _LQ + lq
                acc0 = jnp.zeros((16,), jnp.float32)
                acc1 = jnp.zeros((16,), jnp.float32)
                for corner in range(16):
                    slot = (lq * 16 + corner) * _C + p16
                    w = w_v[seg, pl.ds(slot, 16)]
                    offv = off_v[seg, pl.ds(slot, 16)]
                    rvec = slot + iota
                    f0 = plsc.load_gather(rows, [rvec, offv])
                    f1 = plsc.load_gather(rows, [rvec + jnp.int32(_NQ), offv])
                    acc0 = acc0 + w * f0
                    acc1 = acc1 + w * f1
                plsc.store_scatter(enc_v, [rows_pt, jnp.full((16,), 2 * l, jnp.int32)], acc0)
                plsc.store_scatter(enc_v, [rows_pt, jnp.full((16,), 2 * l + 1, jnp.int32)], acc1)
            return c2

        lax.fori_loop(0, _G, grp_c, 0)

    def chunk_body(k, carry):
        cbase = wid * pts_per_w + k * _C
        pltpu.sync_copy(xyz_hbm.at[pl.ds(cbase * 4, _C * 4)], coords_v)
        phase_a(0)
        d0 = fire(0)
        phase_a(1)
        d1 = fire(1)
        d0.wait()
        phase_c(0)
        phase_a(2)
        d2 = fire(2)
        d1.wait()
        phase_c(1)
        phase_a(3)
        d3 = fire(3)
        d2.wait()
        phase_c(2)
        d3.wait()
        phase_c(3)
        pltpu.sync_copy(enc_v, enc_hbm.at[pl.ds(cbase, _C), :])
        return carry

    lax.fori_loop(0, nchunks, chunk_body, 0)


def _hash_encode_sc(xyz_flat, tab2d, n_pts):
    mesh = plsc.VectorSubcoreMesh(core_axis_name="c", subcore_axis_name="s",
                                  num_cores=_NC, num_subcores=_NS)
    return pl.kernel(
        functools.partial(_enc_body, n_pts=n_pts),
        out_type=jax.ShapeDtypeStruct((n_pts, 2 * _L), jnp.float32),
        mesh=mesh,
        compiler_params=pltpu.CompilerParams(needs_layout_passes=False,
                                             use_tc_tiling_on_sc=False),
        scratch_types=[
            pltpu.VMEM((4 * _C,), jnp.float32),
            pltpu.VMEM((_NSEG, _NIDXH), jnp.int32),
            pltpu.VMEM((_NSEG, _NQ), jnp.float32),
            pltpu.VMEM((_NSEG, _NQ), jnp.int32),
            pltpu.VMEM((2, _NIDXH, 8), jnp.float32),
            pltpu.VMEM((_C, 2 * _L), jnp.float32),
            pltpu.SemaphoreType.DMA,
            pltpu.SemaphoreType.DMA,
        ],
    )(xyz_flat, tab2d)


def _mlp_body(enc_ref, dirs_ref, w0, w1, c0, c1, c2, out_ref):
    e = enc_ref[...]
    dn = (((1,), (0,)), ((), ()))
    h1 = jnp.maximum(
        lax.dot_general(e, w0[...], dn, preferred_element_type=jnp.float32),
        0.0)
    h = lax.dot_general(h1, w1[...], dn, preferred_element_type=jnp.float32)
    d = dirs_ref[...] * 2.0 - 1.0
    x = d[:, 0:1]
    y = d[:, 1:2]
    z = d[:, 2:3]
    x2, y2, z2 = x * x, y * y, z * z
    xy, yz, xz = x * y, y * z, x * z
    de = jnp.concatenate([
        0.28209479177387814 * jnp.ones_like(x),
        -0.48860251190291987 * y,
        0.48860251190291987 * z,
        -0.48860251190291987 * x,
        1.0925484305920792 * xy,
        -1.0925484305920792 * yz,
        0.94617469575755997 * z2 - 0.31539156525252005,
        -1.0925484305920792 * xz,
        0.54627421529603959 * (x2 - y2),
        0.59004358992664352 * y * (3.0 * x2 - y2),
        2.8906114426405538 * xy * z,
        0.45704579946446572 * y * (1.0 - 5.0 * z2),
        0.3731763325901154 * z * (5.0 * z2 - 3.0),
        0.45704579946446572 * x * (1.0 - 5.0 * z2),
        1.4453057213202769 * z * (x2 - y2),
        0.59004358992664352 * x * (x2 - 3.0 * y2),
    ], axis=1)
    cin = jnp.concatenate([de, h], axis=1)
    h2 = jnp.maximum(
        lax.dot_general(cin, c0[...], dn, preferred_element_type=jnp.float32),
        0.0)
    h3 = jnp.maximum(
        lax.dot_general(h2, c1[...], dn, preferred_element_type=jnp.float32),
        0.0)
    co = lax.dot_general(h3, c2[...], dn, preferred_element_type=jnp.float32)
    color = 1.0 / (1.0 + jnp.exp(-co))
    sigma = jnp.exp(h[:, 0:1])
    out_ref[...] = jnp.concatenate([color, sigma], axis=1)


def _mlp_tc(enc, dirs, W0, W1, C0, C1, C2, n_pts):
    bn = 2048
    grid = (n_pts // bn,)
    return pl.pallas_call(
        _mlp_body,
        grid=grid,
        in_specs=[
            pl.BlockSpec((bn, 2 * _L), lambda i: (i, 0)),
            pl.BlockSpec((bn, 3), lambda i: (i, 0)),
            pl.BlockSpec((32, 64), lambda i: (0, 0)),
            pl.BlockSpec((64, 16), lambda i: (0, 0)),
            pl.BlockSpec((32, 64), lambda i: (0, 0)),
            pl.BlockSpec((64, 64), lambda i: (0, 0)),
            pl.BlockSpec((64, 3), lambda i: (0, 0)),
        ],
        out_specs=pl.BlockSpec((bn, 4), lambda i: (i, 0)),
        out_shape=jax.ShapeDtypeStruct((n_pts, 4), jnp.float32),
    )(enc, dirs, W0, W1, C0, C1, C2)


def kernel(original_xyzs, dirs, tables, W0, W1, C0, C1, C2):
    n_pts = original_xyzs.shape[0]
    tab8 = (tables.reshape(_L, _T // 128, 128, _F)
            .swapaxes(2, 3)
            .reshape(_L * _T * _F // 8, 8))
    nsplit = 4
    nh = n_pts // nsplit
    outs = []
    for h in range(nsplit):
        xyz_h = original_xyzs[h * nh:(h + 1) * nh].reshape(-1)
        enc_h = _hash_encode_sc(xyz_h, tab8, nh)
        outs.append(_mlp_tc(enc_h, dirs[h * nh:(h + 1) * nh],
                            W0, W1, C0, C1, C2, nh))
    out = jnp.concatenate(outs, axis=0)
    color = out[:, :3]
    sigma = out[:, 3]
    return (color, sigma)


# speedup vs baseline: 2.5875x; 1.0208x over previous
import functools

import numpy as np
import jax
import jax.numpy as jnp
from jax import lax
from jax.experimental import pallas as pl
from jax.experimental.pallas import tpu as pltpu
from jax.experimental.pallas import tpu_sc as plsc

_L = 16
_F = 2
_T = 2 ** 19
_NMIN = 8
_NMAX = 2048
_BETA = float(np.exp(np.log(_NMAX / _NMIN) / (_L - 1)))
_RES = [int(np.floor(_NMIN * _BETA ** l)) for l in range(_L)]
_DENSE = [(r + 1) ** 4 <= _T for r in _RES]
_HK = [1,
       int(np.uint32(2654435761).astype(np.int32)),
       int(np.uint32(805459861).astype(np.int32)),
       int(np.uint32(3674653429).astype(np.int32))]

_NC = 2
_NS = 16
_NW = _NC * _NS

_C = 32
_G = _C // 16
_NSEG = 8
_LQ = _L // _NSEG
_NQ = _LQ * 16 * _C
_NIDXH = 2 * _NQ


def _enc_body(xyz_hbm, tab_hbm, enc_hbm, coords_v, idx_v, w_v, off_v, rows_v,
              enc_v, sem0, sem1, *, n_pts):
    wid = lax.axis_index("s") * _NC + lax.axis_index("c")
    pts_per_w = n_pts // _NW
    nchunks = pts_per_w // _C
    iota = lax.iota(jnp.int32, 16)
    sems = (sem0, sem1)

    def phase_a(seg):
        def grp_a(g, c2):
            p16 = g * 16
            cvec = []
            for d in range(4):
                v = plsc.load_gather(coords_v, [(p16 + iota) * 4 + d])
                if d < 3:
                    v = v * jnp.float32(1.0 / 3.0) + jnp.float32(0.5)
                cvec.append(v)
            for lq in range(_LQ):
                l = seg * _LQ + lq
                res = _RES[l]
                dense = _DENSE[l]
                r1 = res + 1
                cont_lo, cont_hi, wlo, whi = [], [], [], []
                for d in range(4):
                    scaled = cvec[d] * jnp.float32(res)
                    pos = scaled.astype(jnp.int32)
                    frac = scaled - pos.astype(jnp.float32)
                    a = jnp.clip(pos, 0, res)
                    b = jnp.clip(pos + 1, 0, res)
                    kd = (r1 ** d) if dense else _HK[d]
                    if kd == 1:
                        cont_lo.append(a)
                        cont_hi.append(b)
                    else:
                        kd = jnp.int32(kd)
                        cont_lo.append(a * kd)
                        cont_hi.append(b * kd)
                    wlo.append(jnp.float32(1.0) - frac)
                    whi.append(frac)
                if dense:
                    comb = lambda u, v: u + v
                else:
                    comb = lambda u, v: u ^ v
                h01 = [comb(cont_lo[0], cont_lo[1]), comb(cont_hi[0], cont_lo[1]),
                       comb(cont_lo[0], cont_hi[1]), comb(cont_hi[0], cont_hi[1])]
                w01 = [wlo[0] * wlo[1], whi[0] * wlo[1],
                       wlo[0] * whi[1], whi[0] * whi[1]]
                h23 = [comb(cont_lo[2], cont_lo[3]), comb(cont_hi[2], cont_lo[3]),
                       comb(cont_lo[2], cont_hi[3]), comb(cont_hi[2], cont_hi[3])]
                w23 = [wlo[2] * wlo[3], whi[2] * wlo[3],
                       wlo[2] * whi[3], whi[2] * whi[3]]
                for corner in range(16):
                    i01 = corner & 3
                    i23 = corner >> 2
                    idx = comb(h01[i01], h23[i23])
                    if not dense:
                        idx = idx & jnp.int32(_T - 1)
                    w = w01[i01] * w23[i23]
                    slot = (lq * 16 + corner) * _C + p16
                    w0 = (jnp.int32(l * 2 * _T)
                          + lax.shift_left(lax.shift_right_logical(idx, 7), 8)
                          + (idx & jnp.int32(127)))
                    q0 = lax.shift_right_logical(w0, 3)
                    idx_v[seg, pl.ds(slot, 16)] = q0
                    idx_v[seg, pl.ds(_NQ + slot, 16)] = q0 + jnp.int32(16)
                    off_v[seg, pl.ds(slot, 16)] = idx & jnp.int32(7)
                    w_v[seg, pl.ds(slot, 16)] = w
            return c2

        lax.fori_loop(0, _G, grp_a, 0)

    def fire(seg):
        return pltpu.async_copy(tab_hbm.at[idx_v.at[seg]], rows_v.at[seg % 2],
                                sems[seg % 2])

    def phase_c(seg):
        rows = rows_v.at[seg % 2]

        def grp_c(g, c2):
            p16 = g * 16
            rows_pt = p16 + iota
            for lq in range(_LQ):
                l = seg * _LQ + lq
                acc0 = jnp.zeros((16,), jnp.float32)
                acc1 = jnp.zeros((16,), jnp.float32)
                for corner in range(16):
                    slot = (lq * 16 + corner) * _C + p16
                    w = w_v[seg, pl.ds(slot, 16)]
                    offv = off_v[seg, pl.ds(slot, 16)]
                    rvec = slot + iota
                    f0 = plsc.load_gather(rows, [rvec, offv])
                    f1 = plsc.load_gather(rows, [rvec + jnp.int32(_NQ), offv])
                    acc0 = acc0 + w * f0
                    acc1 = acc1 + w * f1
                plsc.store_scatter(enc_v, [rows_pt, jnp.full((16,), 2 * l, jnp.int32)], acc0)
                plsc.store_scatter(enc_v, [rows_pt, jnp.full((16,), 2 * l + 1, jnp.int32)], acc1)
            return c2

        lax.fori_loop(0, _G, grp_c, 0)

    def chunk_body(k, carry):
        cbase = wid * pts_per_w + k * _C
        pltpu.sync_copy(xyz_hbm.at[pl.ds(cbase * 4, _C * 4)], coords_v)
        cps = [None] * _NSEG
        phase_a(0)
        cps[0] = fire(0)
        phase_a(1)
        cps[1] = fire(1)
        for s in range(2, _NSEG):
            cps[s - 2].wait()
            phase_c(s - 2)
            phase_a(s)
            cps[s] = fire(s)
        cps[_NSEG - 2].wait()
        phase_c(_NSEG - 2)
        cps[_NSEG - 1].wait()
        phase_c(_NSEG - 1)
        pltpu.sync_copy(enc_v, enc_hbm.at[pl.ds(cbase, _C), :])
        return carry

    lax.fori_loop(0, nchunks, chunk_body, 0)


def _hash_encode_sc(xyz_flat, tab2d, n_pts):
    mesh = plsc.VectorSubcoreMesh(core_axis_name="c", subcore_axis_name="s",
                                  num_cores=_NC, num_subcores=_NS)
    return pl.kernel(
        functools.partial(_enc_body, n_pts=n_pts),
        out_type=jax.ShapeDtypeStruct((n_pts, 2 * _L), jnp.float32),
        mesh=mesh,
        compiler_params=pltpu.CompilerParams(needs_layout_passes=False,
                                             use_tc_tiling_on_sc=False),
        scratch_types=[
            pltpu.VMEM((4 * _C,), jnp.float32),
            pltpu.VMEM((_NSEG, _NIDXH), jnp.int32),
            pltpu.VMEM((_NSEG, _NQ), jnp.float32),
            pltpu.VMEM((_NSEG, _NQ), jnp.int32),
            pltpu.VMEM((2, _NIDXH, 8), jnp.float32),
            pltpu.VMEM((_C, 2 * _L), jnp.float32),
            pltpu.SemaphoreType.DMA,
            pltpu.SemaphoreType.DMA,
        ],
    )(xyz_flat, tab2d)


def _mlp_body(enc_ref, dirs_ref, w0, w1, c0, c1, c2, out_ref):
    e = enc_ref[...]
    dn = (((1,), (0,)), ((), ()))
    h1 = jnp.maximum(
        lax.dot_general(e, w0[...], dn, preferred_element_type=jnp.float32),
        0.0)
    h = lax.dot_general(h1, w1[...], dn, preferred_element_type=jnp.float32)
    d = dirs_ref[...] * 2.0 - 1.0
    x = d[:, 0:1]
    y = d[:, 1:2]
    z = d[:, 2:3]
    x2, y2, z2 = x * x, y * y, z * z
    xy, yz, xz = x * y, y * z, x * z
    de = jnp.concatenate([
        0.28209479177387814 * jnp.ones_like(x),
        -0.48860251190291987 * y,
        0.48860251190291987 * z,
        -0.48860251190291987 * x,
        1.0925484305920792 * xy,
        -1.0925484305920792 * yz,
        0.94617469575755997 * z2 - 0.31539156525252005,
        -1.0925484305920792 * xz,
        0.54627421529603959 * (x2 - y2),
        0.59004358992664352 * y * (3.0 * x2 - y2),
        2.8906114426405538 * xy * z,
        0.45704579946446572 * y * (1.0 - 5.0 * z2),
        0.3731763325901154 * z * (5.0 * z2 - 3.0),
        0.45704579946446572 * x * (1.0 - 5.0 * z2),
        1.4453057213202769 * z * (x2 - y2),
        0.59004358992664352 * x * (x2 - 3.0 * y2),
    ], axis=1)
    cin = jnp.concatenate([de, h], axis=1)
    h2 = jnp.maximum(
        lax.dot_general(cin, c0[...], dn, preferred_element_type=jnp.float32),
        0.0)
    h3 = jnp.maximum(
        lax.dot_general(h2, c1[...], dn, preferred_element_type=jnp.float32),
        0.0)
    co = lax.dot_general(h3, c2[...], dn, preferred_element_type=jnp.float32)
    color = 1.0 / (1.0 + jnp.exp(-co))
    sigma = jnp.exp(h[:, 0:1])
    out_ref[...] = jnp.concatenate([color, sigma], axis=1)


def _mlp_tc(enc, dirs, W0, W1, C0, C1, C2, n_pts):
    bn = 2048
    grid = (n_pts // bn,)
    return pl.pallas_call(
        _mlp_body,
        grid=grid,
        in_specs=[
            pl.BlockSpec((bn, 2 * _L), lambda i: (i, 0)),
            pl.BlockSpec((bn, 3), lambda i: (i, 0)),
            pl.BlockSpec((32, 64), lambda i: (0, 0)),
            pl.BlockSpec((64, 16), lambda i: (0, 0)),
            pl.BlockSpec((32, 64), lambda i: (0, 0)),
            pl.BlockSpec((64, 64), lambda i: (0, 0)),
            pl.BlockSpec((64, 3), lambda i: (0, 0)),
        ],
        out_specs=pl.BlockSpec((bn, 4), lambda i: (i, 0)),
        out_shape=jax.ShapeDtypeStruct((n_pts, 4), jnp.float32),
    )(enc, dirs, W0, W1, C0, C1, C2)


def kernel(original_xyzs, dirs, tables, W0, W1, C0, C1, C2):
    n_pts = original_xyzs.shape[0]
    tab8 = (tables.reshape(_L, _T // 128, 128, _F)
            .swapaxes(2, 3)
            .reshape(_L * _T * _F // 8, 8))
    nsplit = 4
    nh = n_pts // nsplit
    outs = []
    for h in range(nsplit):
        xyz_h = original_xyzs[h * nh:(h + 1) * nh].reshape(-1)
        enc_h = _hash_encode_sc(xyz_h, tab8, nh)
        outs.append(_mlp_tc(enc_h, dirs[h * nh:(h + 1) * nh],
                            W0, W1, C0, C1, C2, nh))
    out = jnp.concatenate(outs, axis=0)
    color = out[:, :3]
    sigma = out[:, 3]
    return (color, sigma)


# speedup vs baseline: 2.6132x; 1.0099x over previous
import functools

import numpy as np
import jax
import jax.numpy as jnp
from jax import lax
from jax.experimental import pallas as pl
from jax.experimental.pallas import tpu as pltpu
from jax.experimental.pallas import tpu_sc as plsc

_L = 16
_F = 2
_T = 2 ** 19
_NMIN = 8
_NMAX = 2048
_BETA = float(np.exp(np.log(_NMAX / _NMIN) / (_L - 1)))
_RES = [int(np.floor(_NMIN * _BETA ** l)) for l in range(_L)]
_DENSE = [(r + 1) ** 4 <= _T for r in _RES]
_HK = [1,
       int(np.uint32(2654435761).astype(np.int32)),
       int(np.uint32(805459861).astype(np.int32)),
       int(np.uint32(3674653429).astype(np.int32))]

_NC = 2
_NS = 16
_NW = _NC * _NS

_C = 32
_G = _C // 16
_NSEG = 8
_LQ = _L // _NSEG
_NQ = _LQ * 16 * _C
_NIDXH = 2 * _NQ


def _enc_body(xyz_hbm, tab_hbm, enc_hbm, coords_v, idx_v, w_v, off_v, rows_v,
              enc_v, sem0, sem1, *, n_pts):
    wid = lax.axis_index("s") * _NC + lax.axis_index("c")
    pts_per_w = n_pts // _NW
    nchunks = pts_per_w // _C
    iota = lax.iota(jnp.int32, 16)
    sems = (sem0, sem1)

    def phase_a(seg):
        def grp_a(g, c2):
            p16 = g * 16
            cvec = []
            for d in range(4):
                v = plsc.load_gather(coords_v, [(p16 + iota) * 4 + d])
                if d < 3:
                    v = v * jnp.float32(1.0 / 3.0) + jnp.float32(0.5)
                cvec.append(v)
            for lq in range(_LQ):
                l = seg * _LQ + lq
                res = _RES[l]
                dense = _DENSE[l]
                r1 = res + 1
                cont_lo, cont_hi, wlo, whi = [], [], [], []
                for d in range(4):
                    scaled = cvec[d] * jnp.float32(res)
                    pos = scaled.astype(jnp.int32)
                    frac = scaled - pos.astype(jnp.float32)
                    a = jnp.clip(pos, 0, res)
                    b = jnp.clip(pos + 1, 0, res)
                    kd = (r1 ** d) if dense else _HK[d]
                    if kd == 1:
                        cont_lo.append(a)
                        cont_hi.append(b)
                    else:
                        kd = jnp.int32(kd)
                        cont_lo.append(a * kd)
                        cont_hi.append(b * kd)
                    wlo.append(jnp.float32(1.0) - frac)
                    whi.append(frac)
                if dense:
                    comb = lambda u, v: u + v
                else:
                    comb = lambda u, v: u ^ v
                h01 = [comb(cont_lo[0], cont_lo[1]), comb(cont_hi[0], cont_lo[1]),
                       comb(cont_lo[0], cont_hi[1]), comb(cont_hi[0], cont_hi[1])]
                w01 = [wlo[0] * wlo[1], whi[0] * wlo[1],
                       wlo[0] * whi[1], whi[0] * whi[1]]
                h23 = [comb(cont_lo[2], cont_lo[3]), comb(cont_hi[2], cont_lo[3]),
                       comb(cont_lo[2], cont_hi[3]), comb(cont_hi[2], cont_hi[3])]
                w23 = [wlo[2] * wlo[3], whi[2] * wlo[3],
                       wlo[2] * whi[3], whi[2] * whi[3]]
                for corner in range(16):
                    i01 = corner & 3
                    i23 = corner >> 2
                    idx = comb(h01[i01], h23[i23])
                    if not dense:
                        idx = idx & jnp.int32(_T - 1)
                    w = w01[i01] * w23[i23]
                    slot = (lq * 16 + corner) * _C + p16
                    w0 = (jnp.int32(l * 2 * _T)
                          + lax.shift_left(lax.shift_right_logical(idx, 7), 8)
                          + (idx & jnp.int32(127)))
                    q0 = lax.shift_right_logical(w0, 3)
                    idx_v[seg, pl.ds(slot, 16)] = q0
                    idx_v[seg, pl.ds(_NQ + slot, 16)] = q0 + jnp.int32(16)
                    off_v[seg, pl.ds(slot, 16)] = idx & jnp.int32(7)
                    w_v[seg, pl.ds(slot, 16)] = w
            return c2

        lax.fori_loop(0, _G, grp_a, 0)

    def fire(seg):
        return pltpu.async_copy(tab_hbm.at[idx_v.at[seg]], rows_v.at[seg % 2],
                                sems[seg % 2])

    def phase_c(seg):
        rows = rows_v.at[seg % 2]

        def grp_c(g, c2):
            p16 = g * 16
            rows_pt = p16 + iota
            for lq in range(_LQ):
                l = seg * _LQ + lq
                acc0 = jnp.zeros((16,), jnp.float32)
                acc1 = jnp.zeros((16,), jnp.float32)
                for corner in range(16):
                    slot = (lq * 16 + corner) * _C + p16
                    w = w_v[seg, pl.ds(slot, 16)]
                    offv = off_v[seg, pl.ds(slot, 16)]
                    rvec = slot + iota
                    f0 = plsc.load_gather(rows, [rvec, offv])
                    f1 = plsc.load_gather(rows, [rvec + jnp.int32(_NQ), offv])
                    acc0 = acc0 + w * f0
                    acc1 = acc1 + w * f1
                plsc.store_scatter(enc_v, [rows_pt, jnp.full((16,), 2 * l, jnp.int32)], acc0)
                plsc.store_scatter(enc_v, [rows_pt, jnp.full((16,), 2 * l + 1, jnp.int32)], acc1)
            return c2

        lax.fori_loop(0, _G, grp_c, 0)

    def chunk_body(k, carry):
        cbase = wid * pts_per_w + k * _C
        pltpu.sync_copy(xyz_hbm.at[pl.ds(cbase * 4, _C * 4)], coords_v)
        cps = [None] * _NSEG
        phase_a(0)
        cps[0] = fire(0)
        phase_a(1)
        cps[1] = fire(1)
        for s in range(2, _NSEG):
            cps[s - 2].wait()
            phase_c(s - 2)
            phase_a(s)
            cps[s] = fire(s)
        cps[_NSEG - 2].wait()
        phase_c(_NSEG - 2)
        cps[_NSEG - 1].wait()
        phase_c(_NSEG - 1)
        pltpu.sync_copy(enc_v, enc_hbm.at[pl.ds(cbase, _C), :])
        return carry

    lax.fori_loop(0, nchunks, chunk_body, 0)


def _hash_encode_sc(xyz_flat, tab2d, n_pts):
    mesh = plsc.VectorSubcoreMesh(core_axis_name="c", subcore_axis_name="s",
                                  num_cores=_NC, num_subcores=_NS)
    return pl.kernel(
        functools.partial(_enc_body, n_pts=n_pts),
        out_type=jax.ShapeDtypeStruct((n_pts, 2 * _L), jnp.float32),
        mesh=mesh,
        compiler_params=pltpu.CompilerParams(needs_layout_passes=False,
                                             use_tc_tiling_on_sc=False),
        scratch_types=[
            pltpu.VMEM((4 * _C,), jnp.float32),
            pltpu.VMEM((_NSEG, _NIDXH), jnp.int32),
            pltpu.VMEM((_NSEG, _NQ), jnp.float32),
            pltpu.VMEM((_NSEG, _NQ), jnp.int32),
            pltpu.VMEM((2, _NIDXH, 8), jnp.float32),
            pltpu.VMEM((_C, 2 * _L), jnp.float32),
            pltpu.SemaphoreType.DMA,
            pltpu.SemaphoreType.DMA,
        ],
    )(xyz_flat, tab2d)


def _mlp_body(enc_ref, dirs_ref, w0, w1, c0, c1, c2, out_ref):
    e = enc_ref[...]
    dn = (((1,), (0,)), ((), ()))
    h1 = jnp.maximum(
        lax.dot_general(e, w0[...], dn, preferred_element_type=jnp.float32),
        0.0)
    h = lax.dot_general(h1, w1[...], dn, preferred_element_type=jnp.float32)
    d = dirs_ref[...] * 2.0 - 1.0
    x = d[:, 0:1]
    y = d[:, 1:2]
    z = d[:, 2:3]
    x2, y2, z2 = x * x, y * y, z * z
    xy, yz, xz = x * y, y * z, x * z
    de = jnp.concatenate([
        0.28209479177387814 * jnp.ones_like(x),
        -0.48860251190291987 * y,
        0.48860251190291987 * z,
        -0.48860251190291987 * x,
        1.0925484305920792 * xy,
        -1.0925484305920792 * yz,
        0.94617469575755997 * z2 - 0.31539156525252005,
        -1.0925484305920792 * xz,
        0.54627421529603959 * (x2 - y2),
        0.59004358992664352 * y * (3.0 * x2 - y2),
        2.8906114426405538 * xy * z,
        0.45704579946446572 * y * (1.0 - 5.0 * z2),
        0.3731763325901154 * z * (5.0 * z2 - 3.0),
        0.45704579946446572 * x * (1.0 - 5.0 * z2),
        1.4453057213202769 * z * (x2 - y2),
        0.59004358992664352 * x * (x2 - 3.0 * y2),
    ], axis=1)
    cin = jnp.concatenate([de, h], axis=1)
    h2 = jnp.maximum(
        lax.dot_general(cin, c0[...], dn, preferred_element_type=jnp.float32),
        0.0)
    h3 = jnp.maximum(
        lax.dot_general(h2, c1[...], dn, preferred_element_type=jnp.float32),
        0.0)
    co = lax.dot_general(h3, c2[...], dn, preferred_element_type=jnp.float32)
    color = 1.0 / (1.0 + jnp.exp(-co))
    sigma = jnp.exp(h[:, 0:1])
    out_ref[...] = jnp.concatenate([color, sigma], axis=1)


def _mlp_tc(enc, dirs, W0, W1, C0, C1, C2, n_pts):
    bn = 2048
    grid = (n_pts // bn,)
    return pl.pallas_call(
        _mlp_body,
        grid=grid,
        in_specs=[
            pl.BlockSpec((bn, 2 * _L), lambda i: (i, 0)),
            pl.BlockSpec((bn, 3), lambda i: (i, 0)),
            pl.BlockSpec((32, 64), lambda i: (0, 0)),
            pl.BlockSpec((64, 16), lambda i: (0, 0)),
            pl.BlockSpec((32, 64), lambda i: (0, 0)),
            pl.BlockSpec((64, 64), lambda i: (0, 0)),
            pl.BlockSpec((64, 3), lambda i: (0, 0)),
        ],
        out_specs=pl.BlockSpec((bn, 4), lambda i: (i, 0)),
        out_shape=jax.ShapeDtypeStruct((n_pts, 4), jnp.float32),
    )(enc, dirs, W0, W1, C0, C1, C2)


def kernel(original_xyzs, dirs, tables, W0, W1, C0, C1, C2):
    n_pts = original_xyzs.shape[0]
    tab8 = (tables.reshape(_L, _T // 128, 128, _F)
            .swapaxes(2, 3)
            .reshape(_L * _T * _F // 8, 8))
    nsplit = 8
    nh = n_pts // nsplit
    outs = []
    for h in range(nsplit):
        xyz_h = original_xyzs[h * nh:(h + 1) * nh].reshape(-1)
        enc_h = _hash_encode_sc(xyz_h, tab8, nh)
        outs.append(_mlp_tc(enc_h, dirs[h * nh:(h + 1) * nh],
                            W0, W1, C0, C1, C2, nh))
    out = jnp.concatenate(outs, axis=0)
    color = out[:, :3]
    sigma = out[:, 3]
    return (color, sigma)
